# Initial kernel scaffold; baseline (speedup 1.0000x reference)
import jax
import jax.numpy as jnp
from jax.experimental import pallas as pl


def kernel(x, W_query, keys):
    raise NotImplementedError("write your pallas kernel here")


# speedup vs baseline: 1.8696x; 1.8696x over previous
import functools

import jax
import jax.numpy as jnp
from jax.experimental import pallas as pl

D = 768
NUM_EXPERTS = 64
TOP_K = 2
ROUTER_DIM = 16
TOKENS = 32768

BLOCK = 2048


def _router_block(x_ref, wq_ref, keys_ref, idx_ref, probs_ref, scores_ref):
    x = x_ref[...]
    wq = wq_ref[...]
    keys = keys_ref[...]

    q = jax.lax.dot_general(
        x, wq, (((1,), (1,)), ((), ())),
        preferred_element_type=jnp.float32,
    )
    scores = jax.lax.dot_general(
        q, keys, (((1,), (1,)), ((), ())),
        preferred_element_type=jnp.float32,
    )
    scores_ref[...] = scores

    eidx = jax.lax.broadcasted_iota(jnp.int32, scores.shape, 1)
    m1 = jnp.max(scores, axis=1, keepdims=True)
    i1 = jnp.min(jnp.where(scores == m1, eidx, NUM_EXPERTS),
                 axis=1, keepdims=True)
    masked = jnp.where(eidx == i1, -jnp.inf, scores)
    m2 = jnp.max(masked, axis=1, keepdims=True)
    i2 = jnp.min(jnp.where(masked == m2, eidx, NUM_EXPERTS),
                 axis=1, keepdims=True)

    idx_ref[...] = jnp.concatenate([i1, i2], axis=1)

    e = jnp.exp(m2 - m1)
    denom = 1.0 + e
    probs_ref[...] = jnp.concatenate([1.0 / denom, e / denom], axis=1)


@jax.jit
def kernel(x, W_query, keys):
    n = x.shape[0]
    grid = (n // BLOCK,)
    out_types = (
        jax.ShapeDtypeStruct((n, TOP_K), jnp.int32),
        jax.ShapeDtypeStruct((n, TOP_K), jnp.float32),
        jax.ShapeDtypeStruct((n, NUM_EXPERTS), jnp.float32),
    )
    topk_idx, probs, scores = pl.pallas_call(
        _router_block,
        grid=grid,
        in_specs=[
            pl.BlockSpec((BLOCK, D), lambda i: (i, 0)),
            pl.BlockSpec((ROUTER_DIM, D), lambda i: (0, 0)),
            pl.BlockSpec((NUM_EXPERTS, ROUTER_DIM), lambda i: (0, 0)),
        ],
        out_specs=(
            pl.BlockSpec((BLOCK, TOP_K), lambda i: (i, 0)),
            pl.BlockSpec((BLOCK, TOP_K), lambda i: (i, 0)),
            pl.BlockSpec((BLOCK, NUM_EXPERTS), lambda i: (i, 0)),
        ),
        out_shape=out_types,
    )(x, W_query, keys)
    return topk_idx, probs, scores


# speedup vs baseline: 1.9930x; 1.0660x over previous
import functools

import jax
import jax.numpy as jnp
from jax.experimental import pallas as pl

D = 768
NUM_EXPERTS = 64
TOP_K = 2
ROUTER_DIM = 16
TOKENS = 32768

BLOCK = 4096


def _router_block(x_ref, wq_ref, keys_ref, idx_ref, probs_ref, scores_ref):
    x = x_ref[...]
    wq = wq_ref[...]
    keys = keys_ref[...]

    q = jax.lax.dot_general(
        x, wq, (((1,), (1,)), ((), ())),
        preferred_element_type=jnp.float32,
    )
    scores = jax.lax.dot_general(
        q, keys, (((1,), (1,)), ((), ())),
        preferred_element_type=jnp.float32,
    )
    scores_ref[...] = scores

    eidx = jax.lax.broadcasted_iota(jnp.int32, scores.shape, 1)
    m1 = jnp.max(scores, axis=1, keepdims=True)
    i1 = jnp.min(jnp.where(scores == m1, eidx, NUM_EXPERTS),
                 axis=1, keepdims=True)
    masked = jnp.where(eidx == i1, -jnp.inf, scores)
    m2 = jnp.max(masked, axis=1, keepdims=True)
    i2 = jnp.min(jnp.where(masked == m2, eidx, NUM_EXPERTS),
                 axis=1, keepdims=True)

    idx_ref[...] = jnp.concatenate([i1, i2], axis=1)

    e = jnp.exp(m2 - m1)
    denom = 1.0 + e
    probs_ref[...] = jnp.concatenate([1.0 / denom, e / denom], axis=1)


@jax.jit
def kernel(x, W_query, keys):
    n = x.shape[0]
    grid = (n // BLOCK,)
    out_types = (
        jax.ShapeDtypeStruct((n, TOP_K), jnp.int32),
        jax.ShapeDtypeStruct((n, TOP_K), jnp.float32),
        jax.ShapeDtypeStruct((n, NUM_EXPERTS), jnp.float32),
    )
    topk_idx, probs, scores = pl.pallas_call(
        _router_block,
        grid=grid,
        in_specs=[
            pl.BlockSpec((BLOCK, D), lambda i: (i, 0)),
            pl.BlockSpec((ROUTER_DIM, D), lambda i: (0, 0)),
            pl.BlockSpec((NUM_EXPERTS, ROUTER_DIM), lambda i: (0, 0)),
        ],
        out_specs=(
            pl.BlockSpec((BLOCK, TOP_K), lambda i: (i, 0)),
            pl.BlockSpec((BLOCK, TOP_K), lambda i: (i, 0)),
            pl.BlockSpec((BLOCK, NUM_EXPERTS), lambda i: (i, 0)),
        ),
        out_shape=out_types,
    )(x, W_query, keys)
    return topk_idx, probs, scores
